# baseline (speedup 1.0000x reference)
import functools

import jax
import jax.numpy as jnp
import numpy as np
from jax import lax
from jax.experimental import pallas as pl
from jax.experimental.pallas import tpu as pltpu
from jax.experimental.pallas import tpu_sc as plsc

N = 128
D = N * (N + 1) // 2
NN = N * N


def _sym_index_table() -> np.ndarray:
    iu = np.triu_indices(N)
    idxmat = np.zeros((N, N), dtype=np.int64)
    idxmat[iu] = np.arange(D, dtype=np.int64)
    idxmat = idxmat + idxmat.T - np.diag(np.diag(idxmat))
    return idxmat.reshape(-1).astype(np.int32)


_IDX_TABLE = _sym_index_table()

_INFO = plsc.get_sparse_core_info()
_NC = _INFO.num_cores
_NS = _INFO.num_subcores
_L = _INFO.num_lanes
_NW = _NC * _NS

_UNROLL = 8
_NBUF = 2


def _make_sc_kernel(batch: int):
    rows_per_w = batch // _NW
    pairs = rows_per_w // _NBUF
    mesh = plsc.VectorSubcoreMesh(core_axis_name="c", subcore_axis_name="s")

    @functools.partial(
        pl.kernel,
        mesh=mesh,
        out_type=jax.ShapeDtypeStruct((batch, NN), jnp.float32),
        scratch_types=[
            pltpu.VMEM((NN,), jnp.int32),
            pltpu.VMEM((_NBUF, D), jnp.float32),
            pltpu.VMEM((_NBUF, NN), jnp.float32),
            pltpu.SemaphoreType.DMA,
            pltpu.SemaphoreType.DMA,
            pltpu.SemaphoreType.DMA,
            pltpu.SemaphoreType.DMA,
        ],
        compiler_params=pltpu.CompilerParams(needs_layout_passes=False),
    )
    def sc_gather(x_hbm, idx_hbm, out_hbm, idx_v, x_v, o_v,
                  in_sem0, in_sem1, out_sem0, out_sem1):
        in_sems = (in_sem0, in_sem1)
        out_sems = (out_sem0, out_sem1)
        wid = lax.axis_index("s") * _NC + lax.axis_index("c")
        base_row = wid * rows_per_w
        pltpu.sync_copy(idx_hbm, idx_v)

        for b in range(_NBUF):
            pltpu.async_copy(x_hbm.at[base_row + b], x_v.at[b], in_sems[b])

        def pair_body(k, carry):
            for b in range(_NBUF):
                r = k * _NBUF + b
                row = base_row + r
                pltpu.make_async_copy(
                    x_hbm.at[row], x_v.at[b], in_sems[b]).wait()

                @pl.when(k > 0)
                def _wait_out():
                    pltpu.make_async_copy(
                        o_v.at[b], out_hbm.at[row], out_sems[b]).wait()

                bvec = jnp.full((_L,), b, dtype=jnp.int32)

                def chunk_body(c, carry2):
                    base = c * (_L * _UNROLL)
                    for u in range(_UNROLL):
                        off = base + u * _L
                        ids = idx_v[pl.ds(off, _L)]
                        vals = plsc.load_gather(x_v, [bvec, ids])
                        o_v[b, pl.ds(off, _L)] = vals
                    return carry2

                lax.fori_loop(0, NN // (_L * _UNROLL), chunk_body, 0,
                              unroll=False)
                pltpu.async_copy(o_v.at[b], out_hbm.at[row], out_sems[b])

                @pl.when(r + _NBUF < rows_per_w)
                def _next_in():
                    pltpu.async_copy(
                        x_hbm.at[row + _NBUF], x_v.at[b], in_sems[b])
            return carry

        lax.fori_loop(0, pairs, pair_body, 0, unroll=False)
        for b in range(_NBUF):
            last = base_row + rows_per_w - _NBUF + b
            pltpu.make_async_copy(
                o_v.at[b], out_hbm.at[last], out_sems[b]).wait()

    return sc_gather


def kernel(x):
    batch, d = x.shape
    assert d == D and batch % _NW == 0
    idx = jnp.asarray(_IDX_TABLE)
    out = _make_sc_kernel(batch)(x, idx)
    return out.reshape(batch, N, N)
